# Initial kernel scaffold; baseline (speedup 1.0000x reference)
import jax
import jax.numpy as jnp
from jax.experimental import pallas as pl


def kernel(noun_matrix, subj_verb_cube, obj_verb_cube, word_ids, verb_ids, subj_ids, obj_ids):
    raise NotImplementedError("write your pallas kernel here")


# speedup vs baseline: 8.5836x; 8.5836x over previous
import functools

import jax
import jax.numpy as jnp
from jax import lax
from jax.experimental import pallas as pl
from jax.experimental.pallas import tpu as pltpu
from jax.experimental.pallas import tpu_sc as plsc

NOUN_VOCAB = 100000
VERB_VOCAB = 1000
D = 64
B = 1024
L = 50
V = 20

NC = 2
NS = 16
NW = NC * NS

SENT_PER_W = B // NW
WORDS_PER_W = SENT_PER_W * L
PAIRS_PER_W = SENT_PER_W * V
WCHUNK_SENT = 8
WCHUNK = WCHUNK_SENT * L
N_WCHUNK = SENT_PER_W // WCHUNK_SENT


def _sc_gather_body(noun_hbm, wids_hbm, sids_hbm, oids_hbm,
                    left_hbm, svec_hbm, ovec_hbm,
                    widx_v, wrows_v, pidx_v, prows_v, left_v, sem):
    wid = lax.axis_index("s") * NC + lax.axis_index("c")

    pbase = wid * PAIRS_PER_W
    pltpu.sync_copy(sids_hbm.at[pl.ds(pbase, PAIRS_PER_W)], pidx_v)
    pltpu.async_copy(noun_hbm.at[pidx_v], prows_v, sem).wait()
    pltpu.sync_copy(prows_v, svec_hbm.at[pl.ds(pbase, PAIRS_PER_W)])

    pltpu.sync_copy(oids_hbm.at[pl.ds(pbase, PAIRS_PER_W)], pidx_v)
    pltpu.async_copy(noun_hbm.at[pidx_v], prows_v, sem).wait()
    pltpu.sync_copy(prows_v, ovec_hbm.at[pl.ds(pbase, PAIRS_PER_W)])

    for ch in range(N_WCHUNK):
        base = wid * WORDS_PER_W + ch * WCHUNK
        pltpu.sync_copy(wids_hbm.at[pl.ds(base, WCHUNK)], widx_v)
        pltpu.async_copy(noun_hbm.at[widx_v], wrows_v, sem).wait()
        for s in range(WCHUNK_SENT):
            def body(r, carry):
                a0, a1, a2, a3 = carry
                row = s * L + r
                return (a0 + wrows_v[row, 0:16],
                        a1 + wrows_v[row, 16:32],
                        a2 + wrows_v[row, 32:48],
                        a3 + wrows_v[row, 48:64])
            z = jnp.zeros((16,), jnp.float32)
            a0, a1, a2, a3 = lax.fori_loop(0, L, body, (z, z, z, z))
            srow = ch * WCHUNK_SENT + s
            left_v[srow, 0:16] = a0
            left_v[srow, 16:32] = a1
            left_v[srow, 32:48] = a2
            left_v[srow, 48:64] = a3
    pltpu.sync_copy(left_v, left_hbm.at[pl.ds(wid * SENT_PER_W, SENT_PER_W)])


_sc_gather = functools.partial(
    pl.kernel,
    mesh=plsc.VectorSubcoreMesh(core_axis_name="c", subcore_axis_name="s"),
    out_type=(
        jax.ShapeDtypeStruct((B, D), jnp.float32),
        jax.ShapeDtypeStruct((B * V, D), jnp.float32),
        jax.ShapeDtypeStruct((B * V, D), jnp.float32),
    ),
    scratch_types=[
        pltpu.VMEM((WCHUNK,), jnp.int32),
        pltpu.VMEM((WCHUNK, D), jnp.float32),
        pltpu.VMEM((PAIRS_PER_W,), jnp.int32),
        pltpu.VMEM((PAIRS_PER_W, D), jnp.float32),
        pltpu.VMEM((SENT_PER_W, D), jnp.float32),
        pltpu.SemaphoreType.DMA,
    ],
    compiler_params=pltpu.CompilerParams(use_tc_tiling_on_sc=False),
)(_sc_gather_body)


G = 8


def _tc_body(vids_ref, cube_s_ref, cube_o_ref, svec_ref, ovec_ref,
             left_ref, out_ref):
    blk = pl.program_id(0)
    for g in range(G):
        acc = jnp.zeros((D, D), jnp.float32)
        for p in range(V):
            w = vids_ref[blk * (G * V) + g * V + p]
            ms = cube_s_ref[pl.ds(w * D, D), :]
            mo = cube_o_ref[pl.ds(w * D, D), :]
            s = svec_ref[g, pl.ds(p * D, D)]
            o = ovec_ref[g, pl.ds(p * D, D)]
            acc = acc + ms * s[None, :] + mo * o[None, :]
        row = jnp.sum(acc, axis=1)
        out_ref[g, :] = left_ref[g, :] + row


def kernel(noun_matrix, subj_verb_cube, obj_verb_cube, word_ids, verb_ids,
           subj_ids, obj_ids):
    wids_flat = word_ids.reshape(B * L).astype(jnp.int32)
    sids_flat = subj_ids.reshape(B * V).astype(jnp.int32)
    oids_flat = obj_ids.reshape(B * V).astype(jnp.int32)
    vids_flat = verb_ids.reshape(B * V).astype(jnp.int32)

    leftover, svec, ovec = _sc_gather(noun_matrix, wids_flat, sids_flat,
                                      oids_flat)
    svec2 = svec.reshape(B, V * D)
    ovec2 = ovec.reshape(B, V * D)
    cube_s = subj_verb_cube.reshape(VERB_VOCAB * D, D)
    cube_o = obj_verb_cube.reshape(VERB_VOCAB * D, D)

    grid_spec = pltpu.PrefetchScalarGridSpec(
        num_scalar_prefetch=1,
        grid=(B // G,),
        in_specs=[
            pl.BlockSpec((VERB_VOCAB * D, D), lambda i, v: (0, 0)),
            pl.BlockSpec((VERB_VOCAB * D, D), lambda i, v: (0, 0)),
            pl.BlockSpec((G, V * D), lambda i, v: (i, 0)),
            pl.BlockSpec((G, V * D), lambda i, v: (i, 0)),
            pl.BlockSpec((G, D), lambda i, v: (i, 0)),
        ],
        out_specs=pl.BlockSpec((G, D), lambda i, v: (i, 0)),
    )
    out = pl.pallas_call(
        _tc_body,
        grid_spec=grid_spec,
        out_shape=jax.ShapeDtypeStruct((B, D), jnp.float32),
        compiler_params=pltpu.CompilerParams(
            vmem_limit_bytes=100 * 1024 * 1024,
        ),
    )(vids_flat, cube_s, cube_o, svec2, ovec2, leftover)
    return out
